# baseline (speedup 1.0000x reference)
import functools

import jax
import jax.numpy as jnp
from jax import lax
from jax.experimental import pallas as pl
from jax.experimental.pallas import tpu as pltpu
from jax.experimental.pallas import tpu_sc as plsc

N_NODES = 50000
D = 512
B = 1024
S1 = 25
S2 = 10

NG = B + B * S2
NW = 32
PER_W = NG // NW
GCHUNK = 88
SGRP = 4
SROWS = SGRP * S1
PAD = 104
NSC = PER_W // SGRP


def _sc_gather_sum(xi, gidx, sidx_pad):
  mesh = plsc.VectorSubcoreMesh(
      core_axis_name="c", subcore_axis_name="s", num_cores=2, num_subcores=16)

  @functools.partial(
      pl.kernel,
      out_type=(
          jax.ShapeDtypeStruct((NG, D), jnp.float32),
          jax.ShapeDtypeStruct((NG, D), jnp.float32),
      ),
      mesh=mesh,
      scratch_types=[
          pltpu.VMEM((PER_W,), jnp.int32),
          pltpu.VMEM((NSC * PAD,), jnp.int32),
          pltpu.VMEM((PAD, D), jnp.float32),
          pltpu.VMEM((PAD, D), jnp.float32),
          pltpu.VMEM((2 * SGRP, D), jnp.float32),
          pltpu.SemaphoreType.DMA,
          pltpu.SemaphoreType.DMA,
          pltpu.SemaphoreType.DMA,
          pltpu.SemaphoreType.DMA,
      ],
  )
  def k(xi_hbm, gidx_hbm, sidx_hbm, gout_hbm, sout_hbm,
        gi_v, si_v, buf0, buf1, sums0, gsem0, gsem1, fsem0, fsem1):
    wid = lax.axis_index("s") * 2 + lax.axis_index("c")
    wbase = wid * PER_W
    bufs = (buf0, buf1)
    gsems = (gsem0, gsem1)
    fsems = (fsem0, fsem1)

    pltpu.sync_copy(gidx_hbm.at[pl.ds(wbase, PER_W)], gi_v)
    pltpu.sync_copy(sidx_hbm.at[pl.ds(wid * (NSC * PAD), NSC * PAD)], si_v)

    def gstart(c, b):
      return pltpu.async_copy(
          xi_hbm.at[gi_v.at[pl.ds(c * GCHUNK, GCHUNK)]],
          bufs[b].at[pl.ds(0, GCHUNK)], gsems[b])

    def gout_flush(c, b):
      return pltpu.async_copy(
          bufs[b].at[pl.ds(0, GCHUNK)],
          gout_hbm.at[pl.ds(wbase + c * GCHUNK, GCHUNK)], fsems[b])

    d0 = gstart(0, 0)
    d1 = gstart(1, 1)
    d0.wait()
    f0 = gout_flush(0, 0)
    d1.wait()
    f1 = gout_flush(1, 1)
    f0.wait()
    d2 = gstart(2, 0)
    f1.wait()
    d3 = gstart(3, 1)
    d2.wait()
    f2 = gout_flush(2, 0)
    d3.wait()
    f3 = gout_flush(3, 1)
    f2.wait()
    f3.wait()

    def sgather(c, b):
      return pltpu.async_copy(
          xi_hbm.at[si_v.at[pl.ds(c * PAD, PAD)]], bufs[b], gsems[b])

    def swait(q2, b):
      pltpu.make_async_copy(
          xi_hbm.at[si_v.at[pl.ds(q2 * PAD, PAD)]], bufs[b],
          gsems[b]).wait()

    def accum(b, half):
      buf = bufs[b]

      def col_chunk(c, _, buf=buf, half=half):
        col = pl.ds(c * 16, 16)
        for g in range(SGRP):
          a = [buf[g * S1 + 5 * kk, col] for kk in range(5)]
          for r in range(1, 5):
            for kk in range(5):
              a[kk] = a[kk] + buf[g * S1 + 5 * kk + r, col]
          sums0[half * SGRP + g, col] = (
              (a[0] + a[1]) + (a[2] + a[3])) + a[4]
        return 0

      lax.fori_loop(0, D // 16, col_chunk, 0)

    def fwait():
      pltpu.make_async_copy(
          sums0, sout_hbm.at[pl.ds(wbase, 2 * SGRP)], fsem0).wait()

    sgather(0, 0)
    sgather(1, 1)

    def body(q, _):
      swait(2 * q, 0)

      @pl.when(q >= 1)
      def _():
        fwait()

      accum(0, 0)

      @pl.when(q < NSC // 2 - 1)
      def _():
        sgather(2 * q + 2, 0)

      swait(2 * q + 1, 1)
      accum(1, 1)

      @pl.when(q < NSC // 2 - 1)
      def _():
        sgather(2 * q + 3, 1)

      pltpu.async_copy(
          sums0, sout_hbm.at[pl.ds(wbase + 8 * q, 2 * SGRP)], fsem0)
      return 0

    lax.fori_loop(0, NSC // 2, body, 0)
    fwait()

  return k(xi, gidx, sidx_pad)


def _l2norm(h):
  n2 = jnp.sum(h * h, axis=-1, keepdims=True)
  return h * jnp.where(n2 > 0, lax.rsqrt(n2), 1.0)


def _tc_layers(h0_t, sum_t, h0_n, sum_n, W0a, W0b, W1a, W1b, b0, b1):
  BLK = 128
  NBLK = B // BLK

  def mm(a, b):
    return lax.dot_general(
        a.astype(jnp.bfloat16), b.astype(jnp.bfloat16),
        (((1,), (0,)), ((), ())), preferred_element_type=jnp.float32)

  def body(h0t_r, st_r, h0n_r, sn_r, w0a_r, w0b_r, w1a_r, w1b_r,
           b0_r, b1_r, z_r):
    inv_s1 = 1.0 / S1
    w0a = w0a_r[...]
    w0b = w0b_r[...]
    h1t = mm(h0t_r[...], w0a) + mm(st_r[...] * inv_s1, w0b) + b0_r[...]
    h1t = _l2norm(jnp.maximum(h1t, 0.0))
    h1n = mm(h0n_r[...], w0a) + mm(sn_r[...] * inv_s1, w0b) + b0_r[...]
    h1n = _l2norm(jnp.maximum(h1n, 0.0))
    agg2 = jnp.mean(h1n.reshape(BLK, S2, D), axis=1)
    z = mm(h1t, w1a_r[...]) + mm(agg2, w1b_r[...]) + b1_r[...]
    z_r[...] = _l2norm(jnp.maximum(z, 0.0))

  full = lambda i: (0, 0)
  return pl.pallas_call(
      body,
      grid=(NBLK,),
      in_specs=[
          pl.BlockSpec((BLK, D), lambda i: (i, 0)),
          pl.BlockSpec((BLK, D), lambda i: (i, 0)),
          pl.BlockSpec((BLK * S2, D), lambda i: (i, 0)),
          pl.BlockSpec((BLK * S2, D), lambda i: (i, 0)),
          pl.BlockSpec((D, D), full),
          pl.BlockSpec((D, D), full),
          pl.BlockSpec((D, D), full),
          pl.BlockSpec((D, D), full),
          pl.BlockSpec((1, D), full),
          pl.BlockSpec((1, D), full),
      ],
      out_specs=pl.BlockSpec((BLK, D), lambda i: (i, 0)),
      out_shape=jax.ShapeDtypeStruct((B, D), jnp.float32),
  )(h0_t, sum_t, h0_n, sum_n, W0a, W0b, W1a, W1b, b0, b1)


def kernel(x, nodes, nb1, nb0_t, nb0_n, W0, b0, W1, b1):
  gidx = jnp.concatenate([nodes, nb1.reshape(-1)]).astype(jnp.int32)
  sidx = jnp.concatenate(
      [nb0_t.reshape(-1), nb0_n.reshape(-1)]).astype(jnp.int32)
  sidx2d = sidx.reshape(NW * NSC, SROWS)
  padv = (jnp.arange(NW * NSC, dtype=jnp.int32)[:, None] * (PAD - SROWS)
          + jnp.arange(PAD - SROWS, dtype=jnp.int32)[None, :]) % N_NODES
  sidx_pad = jnp.concatenate([sidx2d, padv], axis=1).reshape(-1)

  gout, sout = _sc_gather_sum(x, gidx, sidx_pad)

  h0_t, h0_n = gout[:B], gout[B:]
  sum_t, sum_n = sout[:B], sout[B:]

  z = _tc_layers(
      h0_t, sum_t, h0_n, sum_n,
      W0[:D], W0[D:], W1[:D], W1[D:],
      b0.reshape(1, D), b1.reshape(1, D),
  )
  return z


# speedup vs baseline: 1.0135x; 1.0135x over previous
import functools

import jax
import jax.numpy as jnp
from jax import lax
from jax.experimental import pallas as pl
from jax.experimental.pallas import tpu as pltpu
from jax.experimental.pallas import tpu_sc as plsc

N_NODES = 50000
D = 512
B = 1024
S1 = 25
S2 = 10

NG = B + B * S2
NW = 32
PER_W = NG // NW
GCHUNK = 88
SGRP = 4
SROWS = SGRP * S1
PAD = 104
NSC = PER_W // SGRP


def _sc_gather_sum(xi, gidx, sidx_pad):
  mesh = plsc.VectorSubcoreMesh(
      core_axis_name="c", subcore_axis_name="s", num_cores=2, num_subcores=16)

  @functools.partial(
      pl.kernel,
      out_type=(
          jax.ShapeDtypeStruct((NG, D), jnp.float32),
          jax.ShapeDtypeStruct((NG, D), jnp.float32),
      ),
      mesh=mesh,
      scratch_types=[
          pltpu.VMEM((PER_W,), jnp.int32),
          pltpu.VMEM((NSC * PAD,), jnp.int32),
          pltpu.VMEM((PAD, D), jnp.float32),
          pltpu.VMEM((PAD, D), jnp.float32),
          pltpu.VMEM((2 * SGRP, D), jnp.float32),
          pltpu.SemaphoreType.DMA,
          pltpu.SemaphoreType.DMA,
          pltpu.SemaphoreType.DMA,
          pltpu.SemaphoreType.DMA,
      ],
  )
  def k(xi_hbm, gidx_hbm, sidx_hbm, gout_hbm, sout_hbm,
        gi_v, si_v, buf0, buf1, sums0, gsem0, gsem1, fsem0, fsem1):
    wid = lax.axis_index("s") * 2 + lax.axis_index("c")
    wbase = wid * PER_W
    bufs = (buf0, buf1)
    gsems = (gsem0, gsem1)
    fsems = (fsem0, fsem1)

    pltpu.sync_copy(gidx_hbm.at[pl.ds(wbase, PER_W)], gi_v)
    pltpu.sync_copy(sidx_hbm.at[pl.ds(wid * (NSC * PAD), NSC * PAD)], si_v)

    def gstart(c, b):
      return pltpu.async_copy(
          xi_hbm.at[gi_v.at[pl.ds(c * GCHUNK, GCHUNK)]],
          bufs[b].at[pl.ds(0, GCHUNK)], gsems[b])

    def gout_flush(c, b):
      return pltpu.async_copy(
          bufs[b].at[pl.ds(0, GCHUNK)],
          gout_hbm.at[pl.ds(wbase + c * GCHUNK, GCHUNK)], fsems[b])

    d0 = gstart(0, 0)
    d1 = gstart(1, 1)
    d0.wait()
    f0 = gout_flush(0, 0)
    d1.wait()
    f1 = gout_flush(1, 1)
    f0.wait()
    d2 = gstart(2, 0)
    f1.wait()
    d3 = gstart(3, 1)
    d2.wait()
    f2 = gout_flush(2, 0)
    d3.wait()
    f3 = gout_flush(3, 1)
    f2.wait()
    f3.wait()

    def sgather(c, b):
      return pltpu.async_copy(
          xi_hbm.at[si_v.at[pl.ds(c * PAD, PAD)]], bufs[b], gsems[b])

    def swait(q2, b):
      pltpu.make_async_copy(
          xi_hbm.at[si_v.at[pl.ds(q2 * PAD, PAD)]], bufs[b],
          gsems[b]).wait()

    def accum(b, half):
      buf = bufs[b]
      for g in range(SGRP):
        def col_chunk(c, _, g=g, buf=buf, half=half):
          col = pl.ds(c * 16, 16)
          a = [buf[g * S1 + 5 * kk, col] for kk in range(5)]
          for r in range(1, 5):
            for kk in range(5):
              a[kk] = a[kk] + buf[g * S1 + 5 * kk + r, col]
          row = half * SGRP + g
          sums0[row, col] = ((a[0] + a[1]) + (a[2] + a[3])) + a[4]
          return 0
        lax.fori_loop(0, D // 16, col_chunk, 0)

    def fwait():
      pltpu.make_async_copy(
          sums0, sout_hbm.at[pl.ds(wbase, 2 * SGRP)], fsem0).wait()

    sgather(0, 0)
    sgather(1, 1)

    def body(q, _):
      swait(2 * q, 0)

      @pl.when(q >= 1)
      def _():
        fwait()

      accum(0, 0)

      @pl.when(q < NSC // 2 - 1)
      def _():
        sgather(2 * q + 2, 0)

      swait(2 * q + 1, 1)
      accum(1, 1)

      @pl.when(q < NSC // 2 - 1)
      def _():
        sgather(2 * q + 3, 1)

      pltpu.async_copy(
          sums0, sout_hbm.at[pl.ds(wbase + 8 * q, 2 * SGRP)], fsem0)
      return 0

    lax.fori_loop(0, NSC // 2, body, 0)
    fwait()

  return k(xi, gidx, sidx_pad)


def _l2norm(h):
  n2 = jnp.sum(h * h, axis=-1, keepdims=True)
  return h * jnp.where(n2 > 0, lax.rsqrt(n2), 1.0)


def _tc_layers(h0_t, sum_t, h0_n, sum_n, W0a, W0b, W1a, W1b, b0, b1):
  BLK = 128
  NBLK = B // BLK

  def body(h0t_r, st_r, h0n_r, sn_r, w0a_r, w0b_r, w1a_r, w1b_r,
           b0_r, b1_r, z_r):
    inv_s1 = 1.0 / S1
    w0a = w0a_r[...]
    w0b = w0b_r[...]
    h1t = h0t_r[...] @ w0a + (st_r[...] * inv_s1) @ w0b + b0_r[...]
    h1t = _l2norm(jnp.maximum(h1t, 0.0))
    h1n = h0n_r[...] @ w0a + (sn_r[...] * inv_s1) @ w0b + b0_r[...]
    h1n = _l2norm(jnp.maximum(h1n, 0.0))
    agg2 = jnp.mean(h1n.reshape(BLK, S2, D), axis=1)
    z = h1t @ w1a_r[...] + agg2 @ w1b_r[...] + b1_r[...]
    z_r[...] = _l2norm(jnp.maximum(z, 0.0))

  full = lambda i: (0, 0)
  return pl.pallas_call(
      body,
      grid=(NBLK,),
      in_specs=[
          pl.BlockSpec((BLK, D), lambda i: (i, 0)),
          pl.BlockSpec((BLK, D), lambda i: (i, 0)),
          pl.BlockSpec((BLK * S2, D), lambda i: (i, 0)),
          pl.BlockSpec((BLK * S2, D), lambda i: (i, 0)),
          pl.BlockSpec((D, D), full),
          pl.BlockSpec((D, D), full),
          pl.BlockSpec((D, D), full),
          pl.BlockSpec((D, D), full),
          pl.BlockSpec((1, D), full),
          pl.BlockSpec((1, D), full),
      ],
      out_specs=pl.BlockSpec((BLK, D), lambda i: (i, 0)),
      out_shape=jax.ShapeDtypeStruct((B, D), jnp.float32),
  )(h0_t, sum_t, h0_n, sum_n, W0a, W0b, W1a, W1b, b0, b1)


def kernel(x, nodes, nb1, nb0_t, nb0_n, W0, b0, W1, b1):
  gidx = jnp.concatenate([nodes, nb1.reshape(-1)]).astype(jnp.int32)
  sidx = jnp.concatenate(
      [nb0_t.reshape(-1), nb0_n.reshape(-1)]).astype(jnp.int32)
  sidx2d = sidx.reshape(NW * NSC, SROWS)
  padv = (jnp.arange(NW * NSC, dtype=jnp.int32)[:, None] * (PAD - SROWS)
          + jnp.arange(PAD - SROWS, dtype=jnp.int32)[None, :]) % N_NODES
  sidx_pad = jnp.concatenate([sidx2d, padv], axis=1).reshape(-1)

  gout, sout = _sc_gather_sum(x, gidx, sidx_pad)

  h0_t, h0_n = gout[:B], gout[B:]
  sum_t, sum_n = sout[:B], sout[B:]

  z = _tc_layers(
      h0_t, sum_t, h0_n, sum_n,
      W0[:D], W0[D:], W1[:D], W1[D:],
      b0.reshape(1, D), b1.reshape(1, D),
  )
  return z
